# baseline (speedup 1.0000x reference)
import functools

import jax
import jax.numpy as jnp
from jax import lax
from jax.experimental import pallas as pl
from jax.experimental.pallas import tpu as pltpu
from jax.experimental.pallas import tpu_sc as plsc

B = 16384
DU = 64
DI = 32
EPS = 1e-06

NC = 2
NS = 16
NW = NC * NS
BPW = B // NW
CH = 128
NCH = BPW // CH


def _gather_body(users2, items2, gu_tab, gi_tab, gu_out, gi_out,
                 uidx, iidx, urows, irows, sem):
    wid = lax.axis_index("s") * NC + lax.axis_index("c")
    base = wid * BPW
    pltpu.sync_copy(users2.at[pl.ds(wid * NCH, NCH)], uidx)
    pltpu.sync_copy(items2.at[pl.ds(wid * NCH, NCH)], iidx)
    copies = []
    for j in range(NCH):
        copies.append(pltpu.async_copy(
            gu_tab.at[uidx.at[j]], urows.at[pl.ds(j * CH, CH)], sem))
        copies.append(pltpu.async_copy(
            gi_tab.at[iidx.at[j]], irows.at[pl.ds(j * CH, CH)], sem))
    for c in copies:
        c.wait()
    pltpu.sync_copy(urows, gu_out.at[pl.ds(base, BPW)])
    pltpu.sync_copy(irows, gi_out.at[pl.ds(base, BPW)])


_sc_gather = functools.partial(
    pl.kernel,
    mesh=plsc.VectorSubcoreMesh(core_axis_name="c", subcore_axis_name="s"),
    out_type=[
        jax.ShapeDtypeStruct((B, DU), jnp.float32),
        jax.ShapeDtypeStruct((B, DI), jnp.float32),
    ],
    scratch_types=[
        pltpu.VMEM((NCH, CH), jnp.int32),
        pltpu.VMEM((NCH, CH), jnp.int32),
        pltpu.VMEM((BPW, DU), jnp.float32),
        pltpu.VMEM((BPW, DI), jnp.float32),
        pltpu.SemaphoreType.DMA,
    ],
    compiler_params=pltpu.CompilerParams(use_tc_tiling_on_sc=False),
)(_gather_body)


SBLK = 2048


def _score_body(gu_ref, gi_ref, o_ref):
    gu = gu_ref[...]
    gi = gi_ref[...]
    s0 = jnp.sum(gu[:, :DI] * gi, axis=1)
    s1 = jnp.sum(gu[:, DI:] * gi, axis=1)
    a0 = jax.nn.sigmoid(s0)
    a1 = jax.nn.sigmoid(s1)
    sum_log = jnp.log(1.0 - a0 + EPS) + jnp.log(1.0 - a1 + EPS)
    o_ref[...] = 1.0 - (-1.0 / (-1.0 + sum_log))


_score = pl.pallas_call(
    _score_body,
    grid=(B // SBLK,),
    in_specs=[pl.BlockSpec((SBLK, DU), lambda i: (i, 0)),
              pl.BlockSpec((SBLK, DI), lambda i: (i, 0))],
    out_specs=pl.BlockSpec((SBLK,), lambda i: (i,)),
    out_shape=jax.ShapeDtypeStruct((B,), jnp.float32),
)


def kernel(users, items, Gu, Gi):
    users2 = users.reshape(NW * NCH, CH)
    items2 = items.reshape(NW * NCH, CH)
    gu_flat, gi_rows = _sc_gather(users2, items2, Gu, Gi)
    xui = _score(gu_flat, gi_rows)
    return (xui, gu_flat.reshape(B, 2, DI), gi_rows)
